# baseline (speedup 1.0000x reference)
import functools

import jax
import jax.numpy as jnp
from jax import lax
from jax.experimental import pallas as pl
from jax.experimental.pallas import tpu as pltpu
from jax.experimental.pallas import tpu_sc as plsc

MAXLEN = 200
VOCAB = 100000
D = 128
BATCH = 1024

NC = 2
NS = 16
NW = NC * NS

N = BATCH * MAXLEN
RW = N // NW
CHUNK = 128
NCHUNK = RW // CHUNK
LANES = 16
SUBV = D // LANES


NBUF = 4


def _body(ids_hbm, tok_hbm, pos_hbm, out_hbm, idx_v, pos_v,
          b0, b1, b2, b3, g0, g1, g2, g3, s0, s1, s2, s3):
    wid = lax.axis_index("s") * NC + lax.axis_index("c")
    base = wid * RW
    bufs = (b0, b1, b2, b3)
    gsems = (g0, g1, g2, g3)
    ssems = (s0, s1, s2, s3)

    pltpu.sync_copy(ids_hbm.at[wid], idx_v)
    pltpu.sync_copy(pos_hbm, pos_v)

    for b in range(NBUF - 1):
        pltpu.async_copy(tok_hbm.at[idx_v.at[b]], bufs[b], gsems[b])

    ngroups = -(-NCHUNK // NBUF)

    @pl.loop(0, ngroups * NBUF, step=NBUF)
    def _group(g):
        for b in range(NBUF):
            c = g + b
            nb = (b + NBUF - 1) % NBUF

            @pl.when((c + NBUF - 1 < NCHUNK) & (c >= 1))
            def _():
                pltpu.make_async_copy(
                    bufs[nb], out_hbm.at[pl.ds(0, CHUNK)], ssems[nb]
                ).wait()

            @pl.when(c + NBUF - 1 < NCHUNK)
            def _():
                pltpu.async_copy(
                    tok_hbm.at[idx_v.at[c + NBUF - 1]], bufs[nb], gsems[nb]
                )

            @pl.when(c < NCHUNK)
            def _():
                buf = bufs[b]
                pltpu.make_async_copy(
                    tok_hbm.at[idx_v.at[c]], buf, gsems[b]
                ).wait()

                @pl.loop(0, CHUNK, unroll=8)
                def _row(i):
                    p = lax.rem(c * CHUNK + i, MAXLEN)
                    for k in range(SUBV):
                        sl = pl.ds(k * LANES, LANES)
                        plsc.addupdate(buf.at[i, sl], pos_v[p, sl])

                pltpu.async_copy(
                    buf, out_hbm.at[pl.ds(base + c * CHUNK, CHUNK)], ssems[b]
                )

    for b in range(NBUF):
        pltpu.make_async_copy(
            bufs[b], out_hbm.at[pl.ds(0, CHUNK)], ssems[b]
        ).wait()


def kernel(input_ids, token_table, pos_table):
    ids = input_ids.astype(jnp.int32).reshape(NW, NCHUNK, CHUNK)
    mesh = plsc.VectorSubcoreMesh(
        core_axis_name="c", subcore_axis_name="s", num_cores=NC, num_subcores=NS
    )
    run = pl.kernel(
        _body,
        out_type=jax.ShapeDtypeStruct((N, D), jnp.float32),
        mesh=mesh,
        scratch_types=[
            pltpu.VMEM((N // CHUNK // NW, CHUNK), jnp.int32),
            pltpu.VMEM((MAXLEN, D), jnp.float32),
        ] + [pltpu.VMEM((CHUNK, D), jnp.float32)] * NBUF
          + [pltpu.SemaphoreType.DMA] * (2 * NBUF),
    )
    out = run(ids, token_table, pos_table)
    return out.reshape(BATCH, MAXLEN, D)


# speedup vs baseline: 1.9762x; 1.9762x over previous
import functools

import jax
import jax.numpy as jnp
from jax import lax
from jax.experimental import pallas as pl
from jax.experimental.pallas import tpu as pltpu
from jax.experimental.pallas import tpu_sc as plsc

MAXLEN = 200
VOCAB = 100000
D = 128
BATCH = 1024

NC = 2
NS = 16
NW = NC * NS

N = BATCH * MAXLEN
RW = N // NW
CHUNK = 128
NCHUNK = RW // CHUNK
LANES = 16
SUBV = D // LANES


NBUF = 4


def _body(ids_hbm, tok_hbm, pos_hbm, out_hbm, idx_v, pos_v,
          b0, b1, b2, b3, g0, g1, g2, g3, s0, s1, s2, s3):
    wid = lax.axis_index("s") * NC + lax.axis_index("c")
    base = wid * RW
    bufs = (b0, b1, b2, b3)
    gsems = (g0, g1, g2, g3)
    ssems = (s0, s1, s2, s3)

    pltpu.sync_copy(ids_hbm.at[wid], idx_v)
    pltpu.sync_copy(pos_hbm, pos_v)

    for b in range(NBUF - 1):
        pltpu.async_copy(tok_hbm.at[idx_v.at[b]], bufs[b], gsems[b])

    ngroups = -(-NCHUNK // NBUF)

    @pl.loop(0, ngroups * NBUF, step=NBUF)
    def _group(g):
        for b in range(NBUF):
            c = g + b
            nb = (b + NBUF - 1) % NBUF

            @pl.when((c + NBUF - 1 < NCHUNK) & (c >= 1))
            def _():
                pltpu.make_async_copy(
                    bufs[nb], out_hbm.at[pl.ds(0, CHUNK)], ssems[nb]
                ).wait()

            @pl.when(c + NBUF - 1 < NCHUNK)
            def _():
                pltpu.async_copy(
                    tok_hbm.at[idx_v.at[c + NBUF - 1]], bufs[nb], gsems[nb]
                )

            @pl.when(c < NCHUNK)
            def _():
                buf = bufs[b]
                pltpu.make_async_copy(
                    tok_hbm.at[idx_v.at[c]], buf, gsems[b]
                ).wait()

                @plsc.parallel_loop(0, CHUNK, unroll=8)
                def _row(i):
                    p = lax.rem(c * CHUNK + i, MAXLEN)
                    for k in range(SUBV):
                        sl = pl.ds(k * LANES, LANES)
                        plsc.addupdate(buf.at[i, sl], pos_v[p, sl])

                pltpu.async_copy(
                    buf, out_hbm.at[pl.ds(base + c * CHUNK, CHUNK)], ssems[b]
                )

    for b in range(NBUF):
        pltpu.make_async_copy(
            bufs[b], out_hbm.at[pl.ds(0, CHUNK)], ssems[b]
        ).wait()


def kernel(input_ids, token_table, pos_table):
    ids = input_ids.astype(jnp.int32).reshape(NW, NCHUNK, CHUNK)
    mesh = plsc.VectorSubcoreMesh(
        core_axis_name="c", subcore_axis_name="s", num_cores=NC, num_subcores=NS
    )
    run = pl.kernel(
        _body,
        out_type=jax.ShapeDtypeStruct((N, D), jnp.float32),
        mesh=mesh,
        scratch_types=[
            pltpu.VMEM((N // CHUNK // NW, CHUNK), jnp.int32),
            pltpu.VMEM((MAXLEN, D), jnp.float32),
        ] + [pltpu.VMEM((CHUNK, D), jnp.float32)] * NBUF
          + [pltpu.SemaphoreType.DMA] * (2 * NBUF),
    )
    out = run(ids, token_table, pos_table)
    return out.reshape(BATCH, MAXLEN, D)
